# baseline (speedup 1.0000x reference)
import dataclasses
import functools

import jax
import jax.numpy as jnp
from jax import lax
from jax.experimental import pallas as pl
from jax.experimental.pallas import tpu as pltpu
from jax.experimental.pallas import tpu_sc as plsc

NC = 2
NS = 16
NW = NC * NS
LANES = 16
D = 128
SEGE = 512
SCHUNK = 64
MAXCH = SEGE // SCHUNK
SCAP = SEGE + 128


def _split_geometry(n):
    split = -(-n // (2 * 128)) * 128
    h_acc = split + 128
    return split, h_acc


def _sc_agg(h, idx3, zrow, ones, n_nodes, with_deg):
    nseg = idx3.shape[1]
    split, h_acc = _split_geometry(n_nodes)
    zrows = h_acc // NS

    mesh = plsc.VectorSubcoreMesh(core_axis_name="c", subcore_axis_name="s")
    out_type = [jax.ShapeDtypeStruct((NC, 2, h_acc, D), jnp.float32)]
    scratch = [
        pltpu.VMEM((2, 2, SEGE), jnp.int32),
        pltpu.VMEM((SCAP,), jnp.int32),
        pltpu.VMEM((SCAP,), jnp.int32),
        pltpu.VMEM((SCHUNK, D), jnp.float32),
        pltpu.VMEM((SCHUNK, D), jnp.float32),
    ]
    scratch += [pltpu.SemaphoreType.DMA for _ in range(6)]
    scratch += [pltpu.VMEM_SHARED((h_acc, D), jnp.float32)]
    if with_deg:
        out_type += [jax.ShapeDtypeStruct((NC, 2, h_acc, D), jnp.float32)]

    cp = pltpu.CompilerParams()
    if "needs_layout_passes" in pltpu.CompilerParams.__dataclass_fields__:
        cp = dataclasses.replace(cp, needs_layout_passes=False)

    @functools.partial(pl.kernel, out_type=out_type, mesh=mesh,
                       scratch_types=scratch, compiler_params=cp)
    def run(*refs):
        if with_deg:
            (h_hbm, idx_hbm, zrow_hbm, ones_hbm, pacc_hbm, pdeg_hbm,
             iseg, bsrc, bdst, r0, r1, is0, is1, s0, s1, t0, t1,
             acc_sh) = refs
        else:
            (h_hbm, idx_hbm, zrow_hbm, ones_hbm, pacc_hbm,
             iseg, bsrc, bdst, r0, r1, is0, is1, s0, s1, t0, t1,
             acc_sh) = refs
        rows = (r0, r1)
        sems = (s0, s1)
        ssems = (t0, t1)
        isems = (is0, is1)
        cid = lax.axis_index("c")
        sid = lax.axis_index("s")
        wid = sid * NC + cid
        zbase = sid * zrows
        iota16 = lax.broadcasted_iota(jnp.int32, (LANES,), 0)
        split_v = jnp.full((LANES,), split, jnp.int32)

        def scat_wait(b):
            pltpu.make_async_copy(rows[b], acc_sh.at[bdst.at[pl.ds(0, SCHUNK)]],
                                  ssems[b]).wait()

        def seg_slice(s, p):
            return idx_hbm.at[wid].at[s].at[pl.ds(p, 2)]

        def pass_loop(p, flush_chunk):
            sp, dp = 1 - p, p
            pltpu.async_copy(seg_slice(0, p), iseg.at[0], isems[0])

            @pl.loop(0, nseg, step=2)
            def _(g):
                for u in range(2):
                    s = g + u
                    pltpu.make_async_copy(seg_slice(s, p), iseg.at[u],
                                          isems[u]).wait()

                    @pl.when(s + 1 < nseg)
                    def _():
                        pltpu.async_copy(seg_slice(s + 1, p), iseg.at[1 - u],
                                         isems[1 - u])

                    cnt = jnp.int32(0)
                    for k in range(SEGE // LANES):
                        dv = iseg[u, dp, pl.ds(k * LANES, LANES)]
                        sv = iseg[u, sp, pl.ds(k * LANES, LANES)]
                        kk, vv = plsc.sort_key_val(dv, sv)
                        bdst[pl.ds(cnt, LANES)] = kk
                        bsrc[pl.ds(cnt, LANES)] = vv
                        m = dv != split_v
                        cnt = cnt + jnp.sum(m.astype(jnp.int32))

                    base = cnt - lax.rem(cnt, LANES)
                    for j in range(6):
                        off = base + j * LANES
                        keep = (off + iota16) < cnt
                        bsrc[pl.ds(off, LANES)] = jnp.where(
                            keep, bsrc[pl.ds(off, LANES)], 0)
                        bdst[pl.ds(off, LANES)] = jnp.where(
                            keep, bdst[pl.ds(off, LANES)], split_v)

                    nch = lax.div(cnt + (SCHUNK - 1), SCHUNK)
                    for c in range(MAXCH):
                        @pl.when(c < nch)
                        def _():
                            flush_chunk(c)

                    @pl.when(nch >= 1)
                    def _():
                        scat_wait(0)

                    @pl.when(nch >= 2)
                    def _():
                        scat_wait(1)

        def agg_chunk(c):
            b = c % 2
            if c >= 2:
                scat_wait(b)
            pltpu.async_copy(h_hbm.at[bsrc.at[pl.ds(c * SCHUNK, SCHUNK)]],
                             rows[b], sems[b]).wait()
            pltpu.async_copy(rows[b],
                             acc_sh.at[bdst.at[pl.ds(c * SCHUNK, SCHUNK)]],
                             ssems[b], add=True)

        def deg_chunk(c):
            b = c % 2
            if c >= 2:
                scat_wait(b)
            pltpu.async_copy(r0,
                             acc_sh.at[bdst.at[pl.ds(c * SCHUNK, SCHUNK)]],
                             ssems[b], add=True)

        for p in range(2):
            pltpu.sync_copy(zrow_hbm, acc_sh.at[pl.ds(zbase, zrows)])
            plsc.subcore_barrier()
            pass_loop(p, agg_chunk)
            plsc.subcore_barrier()
            pltpu.sync_copy(acc_sh.at[pl.ds(zbase, zrows)],
                            pacc_hbm.at[cid, p].at[pl.ds(zbase, zrows)])

        if with_deg:
            pltpu.sync_copy(ones_hbm, r0)
            for p in range(2):
                pltpu.sync_copy(zrow_hbm, acc_sh.at[pl.ds(zbase, zrows)])
                plsc.subcore_barrier()
                pass_loop(p, deg_chunk)
                plsc.subcore_barrier()
                pltpu.sync_copy(acc_sh.at[pl.ds(zbase, zrows)],
                                pdeg_hbm.at[cid, p].at[pl.ds(zbase, zrows)])

    return run(h, idx3, zrow, ones)


def _acc_specs(blk, split, width):
    hb = split // blk
    return pl.BlockSpec((NC, 1, blk, width),
                        lambda i: (0, i // hb, i % hb, 0))


def _tc_layer1(x, pacc, pdeg, Ws1, Wn1, b1, Ws2, Wn2, b2, blk):
    n, d_in = x.shape
    d_h = Ws1.shape[1]
    d_out = Ws2.shape[1]
    split, _ = _split_geometry(n)
    nb = -(-n // blk)

    def body(x_ref, pacc_ref, pdeg_ref, ws1_ref, wn1_ref, b1_ref, ws2_ref,
             wn2_ref, b2_ref, self2_ref, z2_ref):
        acc = pacc_ref[0, 0] + pacc_ref[1, 0]
        deg = pdeg_ref[0, 0, :, 0:1] + pdeg_ref[1, 0, :, 0:1]
        hn = acc / jnp.maximum(deg, 1.0)
        h1 = jnp.dot(x_ref[...], ws1_ref[...],
                     preferred_element_type=jnp.float32)
        h1 += jnp.dot(hn, wn1_ref[...], preferred_element_type=jnp.float32)
        h1 = jnp.maximum(h1 + b1_ref[...], 0.0)
        self2_ref[...] = jnp.dot(h1, ws2_ref[...],
                                 preferred_element_type=jnp.float32) + b2_ref[...]
        z2_ref[...] = jnp.dot(h1, wn2_ref[...],
                              preferred_element_type=jnp.float32)

    return pl.pallas_call(
        body,
        grid=(nb,),
        in_specs=[
            pl.BlockSpec((blk, d_in), lambda i: (i, 0)),
            _acc_specs(blk, split, D),
            _acc_specs(blk, split, D),
            pl.BlockSpec((d_in, d_h), lambda i: (0, 0)),
            pl.BlockSpec((d_in, d_h), lambda i: (0, 0)),
            pl.BlockSpec((d_h,), lambda i: (0,)),
            pl.BlockSpec((d_h, d_out), lambda i: (0, 0)),
            pl.BlockSpec((d_h, d_out), lambda i: (0, 0)),
            pl.BlockSpec((d_out,), lambda i: (0,)),
        ],
        out_specs=[
            pl.BlockSpec((blk, d_out), lambda i: (i, 0)),
            pl.BlockSpec((blk, d_out), lambda i: (i, 0)),
        ],
        out_shape=[
            jax.ShapeDtypeStruct((n, d_out), jnp.float32),
            jax.ShapeDtypeStruct((n, d_out), jnp.float32),
        ],
    )(x, pacc, pdeg, Ws1, Wn1, b1, Ws2, Wn2, b2)


def _tc_layer2(self2, qacc, pdeg, blk):
    n, d_out = self2.shape
    split, _ = _split_geometry(n)
    nb = -(-n // blk)

    def body(self2_ref, qacc_ref, pdeg_ref, out_ref):
        agg = qacc_ref[0, 0] + qacc_ref[1, 0]
        deg = pdeg_ref[0, 0, :, 0:1] + pdeg_ref[1, 0, :, 0:1]
        out_ref[...] = self2_ref[...] + agg / jnp.maximum(deg, 1.0)

    return pl.pallas_call(
        body,
        grid=(nb,),
        in_specs=[
            pl.BlockSpec((blk, d_out), lambda i: (i, 0)),
            _acc_specs(blk, split, D),
            _acc_specs(blk, split, D),
        ],
        out_specs=pl.BlockSpec((blk, d_out), lambda i: (i, 0)),
        out_shape=jax.ShapeDtypeStruct((n, d_out), jnp.float32),
    )(self2, qacc, pdeg)


def kernel(x, edge_index, target_gid_cnt, W_self1, W_neigh1, b1, W_self2,
           W_neigh2, b2):
    n = x.shape[0]
    e = edge_index.shape[1]
    split, h_acc = _split_geometry(n)
    zrows = h_acc // NS
    grp = NW * SEGE * 2
    e_pad = -(-e // grp) * grp
    nseg = e_pad // (NW * SEGE)
    pad = e_pad - e
    src = jnp.concatenate([edge_index[0], jnp.zeros((pad,), jnp.int32)])
    dst = jnp.concatenate([edge_index[1], jnp.full((pad,), -1, jnp.int32)])
    in0 = (dst >= 0) & (dst < split)
    in1 = dst >= split
    dst0 = jnp.where(in0, dst, split)
    dst1 = jnp.where(in1, dst - split, split)
    idx3 = jnp.stack([dst0.reshape(NW, nseg, SEGE),
                      src.reshape(NW, nseg, SEGE),
                      dst1.reshape(NW, nseg, SEGE)], axis=2)
    zrow = jnp.zeros((zrows, D), jnp.float32)
    ones = jnp.ones((SCHUNK, D), jnp.float32)

    pacc, pdeg = _sc_agg(x, idx3, zrow, ones, n, with_deg=True)
    self2, z2 = _tc_layer1(x, pacc, pdeg, W_self1, W_neigh1, b1, W_self2,
                           W_neigh2, b2, blk=640)
    (qacc,) = _sc_agg(z2, idx3, zrow, ones, n, with_deg=False)
    return _tc_layer2(self2, qacc, pdeg, blk=640)


# speedup vs baseline: 1.0009x; 1.0009x over previous
import dataclasses
import functools

import jax
import jax.numpy as jnp
from jax import lax
from jax.experimental import pallas as pl
from jax.experimental.pallas import tpu as pltpu
from jax.experimental.pallas import tpu_sc as plsc

NC = 2
NS = 16
NW = NC * NS
LANES = 16
D = 128
SEGE = 512
SCHUNK = 64
MAXCH = SEGE // SCHUNK
SCAP = SEGE + 128


def _split_geometry(n):
    split = -(-n // (2 * 128)) * 128
    h_acc = split + 128
    return split, h_acc


def _sc_agg(h, idx3, zrow, ones, n_nodes, with_deg):
    nseg = idx3.shape[1]
    split, h_acc = _split_geometry(n_nodes)
    zrows = h_acc // NS

    mesh = plsc.VectorSubcoreMesh(core_axis_name="c", subcore_axis_name="s")
    out_type = [jax.ShapeDtypeStruct((NC, 2, h_acc, D), jnp.float32)]
    scratch = [
        pltpu.VMEM((2, 2, SEGE), jnp.int32),
        pltpu.VMEM((SCAP,), jnp.int32),
        pltpu.VMEM((SCAP,), jnp.int32),
        pltpu.VMEM((SCHUNK, D), jnp.float32),
        pltpu.VMEM((SCHUNK, D), jnp.float32),
    ]
    scratch += [pltpu.SemaphoreType.DMA for _ in range(6)]
    scratch += [pltpu.VMEM_SHARED((h_acc, D), jnp.float32)]
    if with_deg:
        out_type += [jax.ShapeDtypeStruct((NC, 2, h_acc, D), jnp.float32)]

    cp = pltpu.CompilerParams()
    if "needs_layout_passes" in pltpu.CompilerParams.__dataclass_fields__:
        cp = dataclasses.replace(cp, needs_layout_passes=False)

    @functools.partial(pl.kernel, out_type=out_type, mesh=mesh,
                       scratch_types=scratch, compiler_params=cp)
    def run(*refs):
        if with_deg:
            (h_hbm, idx_hbm, zrow_hbm, ones_hbm, pacc_hbm, pdeg_hbm,
             iseg, bsrc, bdst, r0, r1, is0, is1, s0, s1, t0, t1,
             acc_sh) = refs
        else:
            (h_hbm, idx_hbm, zrow_hbm, ones_hbm, pacc_hbm,
             iseg, bsrc, bdst, r0, r1, is0, is1, s0, s1, t0, t1,
             acc_sh) = refs
        rows = (r0, r1)
        sems = (s0, s1)
        ssems = (t0, t1)
        isems = (is0, is1)
        cid = lax.axis_index("c")
        sid = lax.axis_index("s")
        wid = sid * NC + cid
        zbase = sid * zrows
        iota16 = lax.broadcasted_iota(jnp.int32, (LANES,), 0)
        split_v = jnp.full((LANES,), split, jnp.int32)

        def scat_wait(b):
            pltpu.make_async_copy(rows[b], acc_sh.at[bdst.at[pl.ds(0, SCHUNK)]],
                                  ssems[b]).wait()

        def seg_slice(s, p):
            return idx_hbm.at[wid].at[s].at[pl.ds(p, 2)]

        def pass_loop(p, flush_chunks):
            sp, dp = 1 - p, p
            pltpu.async_copy(seg_slice(0, p), iseg.at[0], isems[0])

            @pl.loop(0, nseg, step=2)
            def _(g):
                for u in range(2):
                    s = g + u
                    pltpu.make_async_copy(seg_slice(s, p), iseg.at[u],
                                          isems[u]).wait()

                    @pl.when(s + 1 < nseg)
                    def _():
                        pltpu.async_copy(seg_slice(s + 1, p), iseg.at[1 - u],
                                         isems[1 - u])

                    cnt = jnp.int32(0)
                    for k in range(SEGE // LANES):
                        dv = iseg[u, dp, pl.ds(k * LANES, LANES)]
                        sv = iseg[u, sp, pl.ds(k * LANES, LANES)]
                        kk, vv = plsc.sort_key_val(dv, sv)
                        bdst[pl.ds(cnt, LANES)] = kk
                        bsrc[pl.ds(cnt, LANES)] = vv
                        m = dv != split_v
                        cnt = cnt + jnp.sum(m.astype(jnp.int32))

                    base = cnt - lax.rem(cnt, LANES)
                    for j in range(6):
                        off = base + j * LANES
                        keep = (off + iota16) < cnt
                        bsrc[pl.ds(off, LANES)] = jnp.where(
                            keep, bsrc[pl.ds(off, LANES)], 0)
                        bdst[pl.ds(off, LANES)] = jnp.where(
                            keep, bdst[pl.ds(off, LANES)], split_v)

                    nch = lax.div(cnt + (SCHUNK - 1), SCHUNK)
                    flush_chunks(nch)

        def gath(c):
            return pltpu.make_async_copy(
                h_hbm.at[bsrc.at[pl.ds(c * SCHUNK, SCHUNK)]],
                rows[c % 2], sems[c % 2])

        def scat(c):
            pltpu.async_copy(rows[c % 2],
                             acc_sh.at[bdst.at[pl.ds(c * SCHUNK, SCHUNK)]],
                             ssems[c % 2], add=True)

        def flush_drain(nch):
            @pl.when(nch >= 1)
            def _():
                scat_wait(0)

            @pl.when(nch >= 2)
            def _():
                scat_wait(1)

        def agg_flush(nch):
            for c in range(MAXCH + 1):
                if c < MAXCH:
                    @pl.when(c < nch)
                    def _(c=c):
                        if c >= 2:
                            scat_wait(c % 2)
                        gath(c).start()

                if c >= 1:
                    @pl.when(c - 1 < nch)
                    def _(cc=c - 1):
                        gath(cc).wait()
                        scat(cc)
            flush_drain(nch)

        def deg_flush(nch):
            for c in range(MAXCH):
                @pl.when(c < nch)
                def _(c=c):
                    if c >= 2:
                        scat_wait(c % 2)
                    pltpu.async_copy(r0,
                                     acc_sh.at[bdst.at[pl.ds(c * SCHUNK,
                                                             SCHUNK)]],
                                     ssems[c % 2], add=True)
            flush_drain(nch)

        for p in range(2):
            pltpu.sync_copy(zrow_hbm, acc_sh.at[pl.ds(zbase, zrows)])
            plsc.subcore_barrier()
            pass_loop(p, agg_flush)
            plsc.subcore_barrier()
            pltpu.sync_copy(acc_sh.at[pl.ds(zbase, zrows)],
                            pacc_hbm.at[cid, p].at[pl.ds(zbase, zrows)])

        if with_deg:
            pltpu.sync_copy(ones_hbm, r0)
            for p in range(2):
                pltpu.sync_copy(zrow_hbm, acc_sh.at[pl.ds(zbase, zrows)])
                plsc.subcore_barrier()
                pass_loop(p, deg_flush)
                plsc.subcore_barrier()
                pltpu.sync_copy(acc_sh.at[pl.ds(zbase, zrows)],
                                pdeg_hbm.at[cid, p].at[pl.ds(zbase, zrows)])

    return run(h, idx3, zrow, ones)


def _acc_specs(blk, split, width):
    hb = split // blk
    return pl.BlockSpec((NC, 1, blk, width),
                        lambda i: (0, i // hb, i % hb, 0))


def _tc_layer1(x, pacc, pdeg, Ws1, Wn1, b1, Ws2, Wn2, b2, blk):
    n, d_in = x.shape
    d_h = Ws1.shape[1]
    d_out = Ws2.shape[1]
    split, _ = _split_geometry(n)
    nb = -(-n // blk)

    def body(x_ref, pacc_ref, pdeg_ref, ws1_ref, wn1_ref, b1_ref, ws2_ref,
             wn2_ref, b2_ref, self2_ref, z2_ref):
        acc = pacc_ref[0, 0] + pacc_ref[1, 0]
        deg = pdeg_ref[0, 0, :, 0:1] + pdeg_ref[1, 0, :, 0:1]
        hn = acc / jnp.maximum(deg, 1.0)
        h1 = jnp.dot(x_ref[...], ws1_ref[...],
                     preferred_element_type=jnp.float32)
        h1 += jnp.dot(hn, wn1_ref[...], preferred_element_type=jnp.float32)
        h1 = jnp.maximum(h1 + b1_ref[...], 0.0)
        self2_ref[...] = jnp.dot(h1, ws2_ref[...],
                                 preferred_element_type=jnp.float32) + b2_ref[...]
        z2_ref[...] = jnp.dot(h1, wn2_ref[...],
                              preferred_element_type=jnp.float32)

    return pl.pallas_call(
        body,
        grid=(nb,),
        in_specs=[
            pl.BlockSpec((blk, d_in), lambda i: (i, 0)),
            _acc_specs(blk, split, D),
            _acc_specs(blk, split, D),
            pl.BlockSpec((d_in, d_h), lambda i: (0, 0)),
            pl.BlockSpec((d_in, d_h), lambda i: (0, 0)),
            pl.BlockSpec((d_h,), lambda i: (0,)),
            pl.BlockSpec((d_h, d_out), lambda i: (0, 0)),
            pl.BlockSpec((d_h, d_out), lambda i: (0, 0)),
            pl.BlockSpec((d_out,), lambda i: (0,)),
        ],
        out_specs=[
            pl.BlockSpec((blk, d_out), lambda i: (i, 0)),
            pl.BlockSpec((blk, d_out), lambda i: (i, 0)),
        ],
        out_shape=[
            jax.ShapeDtypeStruct((n, d_out), jnp.float32),
            jax.ShapeDtypeStruct((n, d_out), jnp.float32),
        ],
    )(x, pacc, pdeg, Ws1, Wn1, b1, Ws2, Wn2, b2)


def _tc_layer2(self2, qacc, pdeg, blk):
    n, d_out = self2.shape
    split, _ = _split_geometry(n)
    nb = -(-n // blk)

    def body(self2_ref, qacc_ref, pdeg_ref, out_ref):
        agg = qacc_ref[0, 0] + qacc_ref[1, 0]
        deg = pdeg_ref[0, 0, :, 0:1] + pdeg_ref[1, 0, :, 0:1]
        out_ref[...] = self2_ref[...] + agg / jnp.maximum(deg, 1.0)

    return pl.pallas_call(
        body,
        grid=(nb,),
        in_specs=[
            pl.BlockSpec((blk, d_out), lambda i: (i, 0)),
            _acc_specs(blk, split, D),
            _acc_specs(blk, split, D),
        ],
        out_specs=pl.BlockSpec((blk, d_out), lambda i: (i, 0)),
        out_shape=jax.ShapeDtypeStruct((n, d_out), jnp.float32),
    )(self2, qacc, pdeg)


def kernel(x, edge_index, target_gid_cnt, W_self1, W_neigh1, b1, W_self2,
           W_neigh2, b2):
    n = x.shape[0]
    e = edge_index.shape[1]
    split, h_acc = _split_geometry(n)
    zrows = h_acc // NS
    grp = NW * SEGE * 2
    e_pad = -(-e // grp) * grp
    nseg = e_pad // (NW * SEGE)
    pad = e_pad - e
    src = jnp.concatenate([edge_index[0], jnp.zeros((pad,), jnp.int32)])
    dst = jnp.concatenate([edge_index[1], jnp.full((pad,), -1, jnp.int32)])
    in0 = (dst >= 0) & (dst < split)
    in1 = dst >= split
    dst0 = jnp.where(in0, dst, split)
    dst1 = jnp.where(in1, dst - split, split)
    idx3 = jnp.stack([dst0.reshape(NW, nseg, SEGE),
                      src.reshape(NW, nseg, SEGE),
                      dst1.reshape(NW, nseg, SEGE)], axis=2)
    zrow = jnp.zeros((zrows, D), jnp.float32)
    ones = jnp.ones((SCHUNK, D), jnp.float32)

    pacc, pdeg = _sc_agg(x, idx3, zrow, ones, n, with_deg=True)
    self2, z2 = _tc_layer1(x, pacc, pdeg, W_self1, W_neigh1, b1, W_self2,
                           W_neigh2, b2, blk=640)
    (qacc,) = _sc_agg(z2, idx3, zrow, ones, n, with_deg=False)
    return _tc_layer2(self2, qacc, pdeg, blk=640)


# speedup vs baseline: 1.4312x; 1.4298x over previous
import functools

import jax
import jax.numpy as jnp
from jax import lax
from jax.experimental import pallas as pl
from jax.experimental.pallas import tpu as pltpu
from jax.experimental.pallas import tpu_sc as plsc

NC = 2
NS = 16
NW = NC * NS
LANES = 16
CHUNK = 32
NBUF = 4
D = 128


def _split_geometry(n):
    split = -(-n // (2 * 128)) * 128
    h_acc = split + 128
    return split, h_acc


def _sc_agg(h, idx4, zrow, ones, n_nodes, with_deg):
    cpt = idx4.shape[1]
    ngrp = cpt // NBUF
    split, h_acc = _split_geometry(n_nodes)
    zrows = h_acc // NS

    mesh = plsc.VectorSubcoreMesh(core_axis_name="c", subcore_axis_name="s")
    out_type = [jax.ShapeDtypeStruct((NC, 2, h_acc, D), jnp.float32)]
    scratch = [pltpu.VMEM((2, NBUF, 3, CHUNK), jnp.int32)]
    scratch += [pltpu.VMEM((CHUNK, D), jnp.float32) for _ in range(NBUF)]
    scratch += [pltpu.SemaphoreType.DMA for _ in range(2 + 2 * NBUF)]
    scratch += [pltpu.VMEM_SHARED((h_acc, D), jnp.float32)]
    if with_deg:
        out_type += [jax.ShapeDtypeStruct((NC, 2, h_acc, D), jnp.float32)]

    @functools.partial(pl.kernel, out_type=out_type, mesh=mesh,
                       scratch_types=scratch)
    def run(*refs):
        if with_deg:
            (h_hbm, idx_hbm, zrow_hbm, ones_hbm, pacc_hbm, pdeg_hbm,
             ibuf, r0, r1, r2, r3, is0, is1, s0, s1, s2, s3, t0, t1, t2, t3, acc_sh) = refs
        else:
            (h_hbm, idx_hbm, zrow_hbm, ones_hbm, pacc_hbm,
             ibuf, r0, r1, r2, r3, is0, is1, s0, s1, s2, s3, t0, t1, t2, t3, acc_sh) = refs
        rows = (r0, r1, r2, r3)
        sems = (s0, s1, s2, s3)
        ssems = (t0, t1, t2, t3)
        isems = (is0, is1)
        cid = lax.axis_index("c")
        sid = lax.axis_index("s")
        wid = sid * NC + cid
        zbase = sid * zrows

        def scat_wait(b):
            pltpu.make_async_copy(rows[b], acc_sh.at[ibuf.at[0, b, 1]],
                                  ssems[b]).wait()

        def idx_loop(pass_body):
            pltpu.async_copy(idx_hbm.at[wid].at[pl.ds(0, NBUF)],
                             ibuf.at[0], isems[0])

            @pl.loop(0, ngrp, step=2)
            def _(g):
                for q in range(2):
                    geff = g + q
                    pltpu.make_async_copy(
                        idx_hbm.at[wid].at[pl.ds(geff * NBUF, NBUF)],
                        ibuf.at[q], isems[q]).wait()

                    @pl.when(geff > 0)
                    def _():
                        for b in range(NBUF):
                            scat_wait(b)

                    @pl.when(geff + 1 < ngrp)
                    def _():
                        pltpu.async_copy(
                            idx_hbm.at[wid].at[pl.ds((geff + 1) * NBUF, NBUF)],
                            ibuf.at[1 - q], isems[1 - q])

                    pass_body(q)

            for b in range(NBUF):
                scat_wait(b)

        for p in range(2):
            pltpu.sync_copy(zrow_hbm, acc_sh.at[pl.ds(zbase, zrows)])
            plsc.subcore_barrier()

            def agg_body(q):
                cps = []
                for b in range(NBUF):
                    cps.append(pltpu.async_copy(
                        h_hbm.at[ibuf.at[q, b, 0]], rows[b], sems[b]))
                for b in range(NBUF):
                    cps[b].wait()
                    pltpu.async_copy(rows[b], acc_sh.at[ibuf.at[q, b, 1 + p]],
                                     ssems[b], add=True)

            idx_loop(agg_body)
            plsc.subcore_barrier()

            pltpu.sync_copy(acc_sh.at[pl.ds(zbase, zrows)],
                            pacc_hbm.at[cid, p].at[pl.ds(zbase, zrows)])

        if with_deg:
            pltpu.sync_copy(ones_hbm, r0)
            for p in range(2):
                pltpu.sync_copy(zrow_hbm, acc_sh.at[pl.ds(zbase, zrows)])
                plsc.subcore_barrier()

                def deg_body(q):
                    for b in range(NBUF):
                        pltpu.async_copy(r0, acc_sh.at[ibuf.at[q, b, 1 + p]],
                                         ssems[b], add=True)

                idx_loop(deg_body)
                plsc.subcore_barrier()
                pltpu.sync_copy(acc_sh.at[pl.ds(zbase, zrows)],
                                pdeg_hbm.at[cid, p].at[pl.ds(zbase, zrows)])

    return run(h, idx4, zrow, ones)


def _acc_specs(blk, split, width):
    hb = split // blk
    return pl.BlockSpec((NC, 1, blk, width),
                        lambda i: (0, i // hb, i % hb, 0))


def _tc_layer1(x, pacc, pdeg, Ws1, Wn1, b1, Ws2, Wn2, b2, blk):
    n, d_in = x.shape
    d_h = Ws1.shape[1]
    d_out = Ws2.shape[1]
    split, _ = _split_geometry(n)
    nb = -(-n // blk)

    def body(x_ref, pacc_ref, pdeg_ref, ws1_ref, wn1_ref, b1_ref, ws2_ref,
             wn2_ref, b2_ref, self2_ref, z2_ref):
        acc = pacc_ref[0, 0] + pacc_ref[1, 0]
        deg = pdeg_ref[0, 0, :, 0:1] + pdeg_ref[1, 0, :, 0:1]
        hn = acc / jnp.maximum(deg, 1.0)
        h1 = jnp.dot(x_ref[...], ws1_ref[...],
                     preferred_element_type=jnp.float32)
        h1 += jnp.dot(hn, wn1_ref[...], preferred_element_type=jnp.float32)
        h1 = jnp.maximum(h1 + b1_ref[...], 0.0)
        self2_ref[...] = jnp.dot(h1, ws2_ref[...],
                                 preferred_element_type=jnp.float32) + b2_ref[...]
        z2_ref[...] = jnp.dot(h1, wn2_ref[...],
                              preferred_element_type=jnp.float32)

    return pl.pallas_call(
        body,
        grid=(nb,),
        in_specs=[
            pl.BlockSpec((blk, d_in), lambda i: (i, 0)),
            _acc_specs(blk, split, D),
            _acc_specs(blk, split, D),
            pl.BlockSpec((d_in, d_h), lambda i: (0, 0)),
            pl.BlockSpec((d_in, d_h), lambda i: (0, 0)),
            pl.BlockSpec((d_h,), lambda i: (0,)),
            pl.BlockSpec((d_h, d_out), lambda i: (0, 0)),
            pl.BlockSpec((d_h, d_out), lambda i: (0, 0)),
            pl.BlockSpec((d_out,), lambda i: (0,)),
        ],
        out_specs=[
            pl.BlockSpec((blk, d_out), lambda i: (i, 0)),
            pl.BlockSpec((blk, d_out), lambda i: (i, 0)),
        ],
        out_shape=[
            jax.ShapeDtypeStruct((n, d_out), jnp.float32),
            jax.ShapeDtypeStruct((n, d_out), jnp.float32),
        ],
    )(x, pacc, pdeg, Ws1, Wn1, b1, Ws2, Wn2, b2)


def _tc_layer2(self2, qacc, pdeg, blk):
    n, d_out = self2.shape
    split, _ = _split_geometry(n)
    nb = -(-n // blk)

    def body(self2_ref, qacc_ref, pdeg_ref, out_ref):
        agg = qacc_ref[0, 0] + qacc_ref[1, 0]
        deg = pdeg_ref[0, 0, :, 0:1] + pdeg_ref[1, 0, :, 0:1]
        out_ref[...] = self2_ref[...] + agg / jnp.maximum(deg, 1.0)

    return pl.pallas_call(
        body,
        grid=(nb,),
        in_specs=[
            pl.BlockSpec((blk, d_out), lambda i: (i, 0)),
            _acc_specs(blk, split, D),
            _acc_specs(blk, split, D),
        ],
        out_specs=pl.BlockSpec((blk, d_out), lambda i: (i, 0)),
        out_shape=jax.ShapeDtypeStruct((n, d_out), jnp.float32),
    )(self2, qacc, pdeg)


def kernel(x, edge_index, target_gid_cnt, W_self1, W_neigh1, b1, W_self2,
           W_neigh2, b2):
    n = x.shape[0]
    e = edge_index.shape[1]
    split, h_acc = _split_geometry(n)
    zrows = h_acc // NS
    grp = NW * CHUNK * NBUF * 2
    e_pad = -(-e // grp) * grp
    cpt = e_pad // (NW * CHUNK)
    pad = e_pad - e
    src = jnp.concatenate([edge_index[0], jnp.zeros((pad,), jnp.int32)])
    dst = jnp.concatenate([edge_index[1], jnp.full((pad,), n, jnp.int32)])
    dst0 = jnp.where(dst < split, dst, split)
    dst1 = jnp.where(dst >= split, dst - split, split)
    idx4 = jnp.stack([src.reshape(NW, cpt, CHUNK),
                      dst0.reshape(NW, cpt, CHUNK),
                      dst1.reshape(NW, cpt, CHUNK)], axis=2)
    zrow = jnp.zeros((zrows, D), jnp.float32)
    ones = jnp.ones((CHUNK, D), jnp.float32)

    pacc, pdeg = _sc_agg(x, idx4, zrow, ones, n, with_deg=True)
    self2, z2 = _tc_layer1(x, pacc, pdeg, W_self1, W_neigh1, b1, W_self2,
                           W_neigh2, b2, blk=640)
    (qacc,) = _sc_agg(z2, idx4, zrow, ones, n, with_deg=False)
    return _tc_layer2(self2, qacc, pdeg, blk=640)
